# baseline (speedup 1.0000x reference)
import numpy as np
import jax
import jax.numpy as jnp
from jax.experimental import pallas as pl
from jax.experimental.pallas import tpu as pltpu

_REMAP = np.array([1, 2, 3, 4, 5, 6, 7, 8, 9, 10, 11, 13, 14, 15, 16, 17,
                   18, 19, 20, 21, 22, 23, 24, 25, 27, 28, 31, 32, 33, 34,
                   35, 36, 37, 38, 39, 40, 41, 42, 43, 44, 46, 47, 48, 49,
                   50, 51, 52, 53, 54, 55, 56, 57, 58, 59, 60, 61, 62, 63,
                   64, 65, 67, 70, 72, 73, 74, 75, 76, 77, 78, 79, 80, 81,
                   82, 84, 85, 86, 87, 88, 89, 90], dtype=np.int32)

_KEEP = np.zeros((91,), dtype=bool)
_KEEP[_REMAP] = True
_R = np.zeros((95, 84), dtype=np.float32)
_R[_REMAP, np.arange(80)] = 1.0
_R[91:95, 80:84] = np.eye(4, dtype=np.float32)

_Q = 900
_QP = 1024
_K = 300
_C = 91


def _body(off_ref, keep_ref, r_ref, lg_ref, bx_ref, outl_ref, outb_ref):
    nb = lg_ref.shape[0]
    lg3 = lg_ref[...]
    keep = keep_ref[...].reshape(1, 1, _C) > 0
    masked = jnp.where(keep, lg3, -jnp.inf)
    vals = jnp.max(masked, axis=2)
    keys = jnp.concatenate(
        [vals, jnp.full((nb, _QP - _Q), -jnp.inf, jnp.float32)], axis=1)

    idxs = jax.lax.broadcasted_iota(jnp.int32, (nb, _QP), 1)
    pos = idxs
    for kk in (2, 4, 8, 16, 32, 64, 128, 256, 512, 1024):
        asc = (pos & kk) != 0
        jj = kk // 2
        while jj >= 1:
            upper = (pos & jj) != 0
            pk = jnp.where(upper, pltpu.roll(keys, jj, 1),
                           pltpu.roll(keys, _QP - jj, 1))
            pi = jnp.where(upper, pltpu.roll(idxs, jj, 1),
                           pltpu.roll(idxs, _QP - jj, 1))
            before = (keys > pk) | ((keys == pk) & (idxs < pi))
            take = before ^ upper ^ asc
            keys = jnp.where(take, keys, pk)
            idxs = jnp.where(take, idxs, pi)
            jj //= 2

    sidx = jnp.transpose(idxs[:, :_K]).astype(jnp.float32) + off_ref[0]
    jiota = jax.lax.broadcasted_iota(jnp.int32, (_K, _Q), 1).astype(jnp.float32)

    data = jnp.concatenate([lg3, bx_ref[...]], axis=2)
    da = data.astype(jnp.bfloat16)
    r1 = data - da.astype(jnp.float32)
    db = r1.astype(jnp.bfloat16)
    rm = r_ref[...].astype(jnp.bfloat16)
    dar = jnp.dot(da.reshape(nb * _Q, _C + 4), rm,
                  preferred_element_type=jnp.float32
                  ).astype(jnp.bfloat16).reshape(nb, _Q, 84)
    dbr = jnp.dot(db.reshape(nb * _Q, _C + 4), rm,
                  preferred_element_type=jnp.float32
                  ).astype(jnp.bfloat16).reshape(nb, _Q, 84)
    dcomb = jnp.concatenate([dar, dbr], axis=2)
    for b in range(nb):
        sel2 = (jiota == sidx[:, b:b + 1]).astype(jnp.bfloat16)
        gc = jnp.dot(sel2, dcomb[b], preferred_element_type=jnp.float32)
        g = gc[:, :84] + gc[:, 84:168]
        outl_ref[b] = g[:, :80]
        outb_ref[b] = g[:, 80:84]


def kernel(pred_logits, pred_boxes, student_num_queries):
    bs = pred_logits.shape[0]
    nb = 16 if bs % 16 == 0 else (8 if bs % 8 == 0 else 1)
    off = (jnp.asarray(student_num_queries, jnp.int32) - _K).astype(jnp.float32)
    off = off.reshape(1)
    grid_spec = pl.GridSpec(
        grid=(bs // nb,),
        in_specs=[
            pl.BlockSpec(memory_space=pltpu.SMEM),
            pl.BlockSpec((1, _C), lambda b: (0, 0)),
            pl.BlockSpec((_C + 4, 84), lambda b: (0, 0)),
            pl.BlockSpec((nb, _Q, _C), lambda b: (b, 0, 0)),
            pl.BlockSpec((nb, _Q, 4), lambda b: (b, 0, 0)),
        ],
        out_specs=[
            pl.BlockSpec((nb, _K, 80), lambda b: (b, 0, 0)),
            pl.BlockSpec((nb, _K, 4), lambda b: (b, 0, 0)),
        ],
    )
    return pl.pallas_call(
        _body,
        grid_spec=grid_spec,
        out_shape=[
            jax.ShapeDtypeStruct((bs, _K, 80), jnp.float32),
            jax.ShapeDtypeStruct((bs, _K, 4), jnp.float32),
        ],
    )(off, jnp.asarray(_KEEP, jnp.float32).reshape(1, _C),
      jnp.asarray(_R), pred_logits, pred_boxes)


# speedup vs baseline: 1.2577x; 1.2577x over previous
import numpy as np
import jax
import jax.numpy as jnp
from jax.experimental import pallas as pl
from jax.experimental.pallas import tpu as pltpu

_REMAP = np.array([1, 2, 3, 4, 5, 6, 7, 8, 9, 10, 11, 13, 14, 15, 16, 17,
                   18, 19, 20, 21, 22, 23, 24, 25, 27, 28, 31, 32, 33, 34,
                   35, 36, 37, 38, 39, 40, 41, 42, 43, 44, 46, 47, 48, 49,
                   50, 51, 52, 53, 54, 55, 56, 57, 58, 59, 60, 61, 62, 63,
                   64, 65, 67, 70, 72, 73, 74, 75, 76, 77, 78, 79, 80, 81,
                   82, 84, 85, 86, 87, 88, 89, 90], dtype=np.int32)

_KEEP = np.zeros((91,), dtype=bool)
_KEEP[_REMAP] = True
_R = np.zeros((95, 84), dtype=np.float32)
_R[_REMAP, np.arange(80)] = 1.0
_R[91:95, 80:84] = np.eye(4, dtype=np.float32)

_Q = 900
_QP = 1024
_K = 300
_C = 91


def _body(off_ref, keep_ref, r_ref, lg_ref, bx_ref, outl_ref, outb_ref):
    nb = lg_ref.shape[0]
    lg3 = lg_ref[...]
    keep = keep_ref[...].reshape(1, 1, _C) > 0
    masked = jnp.where(keep, lg3, -jnp.inf)
    vals = jnp.max(masked, axis=2)
    keys = jnp.concatenate(
        [vals, jnp.full((nb, _QP - _Q), -jnp.inf, jnp.float32)], axis=1)

    idxs = jax.lax.broadcasted_iota(jnp.int32, (nb, _QP), 1)
    pos = idxs
    for kk in (2, 4, 8, 16, 32, 64, 128, 256, 512, 1024):
        asc = (pos & kk) != 0
        jj = kk // 2
        while jj >= 1:
            upper = (pos & jj) != 0
            pk = jnp.where(upper, pltpu.roll(keys, jj, 1),
                           pltpu.roll(keys, _QP - jj, 1))
            pi = jnp.where(upper, pltpu.roll(idxs, jj, 1),
                           pltpu.roll(idxs, _QP - jj, 1))
            before = (keys > pk) | ((keys == pk) & (idxs < pi))
            take = before ^ upper ^ asc
            keys = jnp.where(take, keys, pk)
            idxs = jnp.where(take, idxs, pi)
            jj //= 2

    sidx = jnp.transpose(idxs[:, :_K]).astype(jnp.float32) + off_ref[0]
    jiota = jax.lax.broadcasted_iota(jnp.int32, (_K, _Q), 1).astype(jnp.float32)

    data = jnp.concatenate([lg3, bx_ref[...]], axis=2)
    da = data.astype(jnp.bfloat16)
    r1 = data - da.astype(jnp.float32)
    db = r1.astype(jnp.bfloat16)
    rm = r_ref[...].astype(jnp.bfloat16)
    dar = jnp.dot(da.reshape(nb * _Q, _C + 4), rm,
                  preferred_element_type=jnp.float32
                  ).astype(jnp.bfloat16).reshape(nb, _Q, 84)
    dbr = jnp.dot(db.reshape(nb * _Q, _C + 4), rm,
                  preferred_element_type=jnp.float32
                  ).astype(jnp.bfloat16).reshape(nb, _Q, 84)
    for b in range(nb):
        sel2 = (jiota == sidx[:, b:b + 1]).astype(jnp.bfloat16)
        g = (jnp.dot(sel2, dar[b], preferred_element_type=jnp.float32)
             + jnp.dot(sel2, dbr[b], preferred_element_type=jnp.float32))
        outl_ref[b] = g[:, :80]
        outb_ref[b] = g[:, 80:84]


def kernel(pred_logits, pred_boxes, student_num_queries):
    bs = pred_logits.shape[0]
    nb = 16 if bs % 16 == 0 else (8 if bs % 8 == 0 else 1)
    off = (jnp.asarray(student_num_queries, jnp.int32) - _K).astype(jnp.float32)
    off = off.reshape(1)
    grid_spec = pl.GridSpec(
        grid=(bs // nb,),
        in_specs=[
            pl.BlockSpec(memory_space=pltpu.SMEM),
            pl.BlockSpec((1, _C), lambda b: (0, 0)),
            pl.BlockSpec((_C + 4, 84), lambda b: (0, 0)),
            pl.BlockSpec((nb, _Q, _C), lambda b: (b, 0, 0)),
            pl.BlockSpec((nb, _Q, 4), lambda b: (b, 0, 0)),
        ],
        out_specs=[
            pl.BlockSpec((nb, _K, 80), lambda b: (b, 0, 0)),
            pl.BlockSpec((nb, _K, 4), lambda b: (b, 0, 0)),
        ],
    )
    return pl.pallas_call(
        _body,
        grid_spec=grid_spec,
        out_shape=[
            jax.ShapeDtypeStruct((bs, _K, 80), jnp.float32),
            jax.ShapeDtypeStruct((bs, _K, 4), jnp.float32),
        ],
    )(off, jnp.asarray(_KEEP, jnp.float32).reshape(1, _C),
      jnp.asarray(_R), pred_logits, pred_boxes)
